# Initial kernel scaffold; baseline (speedup 1.0000x reference)
import jax
import jax.numpy as jnp
from jax.experimental import pallas as pl


def kernel(x, pe):
    raise NotImplementedError("write your pallas kernel here")


# speedup vs baseline: 1.0246x; 1.0246x over previous
import functools

import jax
import jax.numpy as jnp
from jax import lax
from jax.experimental import pallas as pl
from jax.experimental.pallas import tpu as pltpu
from jax.experimental.pallas import tpu_sc as plsc

D_MODEL = 2048
MAX_LEN = 8192

NC = 2
NS = 16
NW = NC * NS

CHUNK = 16
NBUF = 2


def _gather_body(steps, table_hbm, idx_hbm, out_hbm, idx_v, rows_v, *sems):
    gsems = sems[:NBUF]
    wsems = sems[NBUF:]
    wid = lax.axis_index("s") * NC + lax.axis_index("c")
    base = wid * (steps * CHUNK)

    pltpu.sync_copy(idx_hbm.at[wid], idx_v)

    for b in range(NBUF):
        pltpu.async_copy(table_hbm.at[idx_v.at[b]], rows_v.at[b], gsems[b])

    @pl.loop(0, steps, step=NBUF)
    def _(g):
        for b in range(NBUF):
            s = g + b
            pltpu.make_async_copy(
                table_hbm.at[idx_v.at[s]], rows_v.at[b], gsems[b]
            ).wait()
            pltpu.async_copy(
                rows_v.at[b], out_hbm.at[pl.ds(base + s * CHUNK, CHUNK)], wsems[b]
            )

            @pl.when(s + NBUF < steps)
            def _():
                pltpu.make_async_copy(
                    rows_v.at[b],
                    out_hbm.at[pl.ds(base + s * CHUNK, CHUNK)],
                    wsems[b],
                ).wait()
                pltpu.async_copy(
                    table_hbm.at[idx_v.at[s + NBUF]], rows_v.at[b], gsems[b]
                )

    for b in range(NBUF):
        s = steps - NBUF + b
        pltpu.make_async_copy(
            rows_v.at[b], out_hbm.at[pl.ds(base + s * CHUNK, CHUNK)], wsems[b]
        ).wait()


@functools.partial(jax.jit, static_argnums=(2,))
def _sc_gather(table, idx, n):
    steps = n // (NW * CHUNK)
    mesh = plsc.VectorSubcoreMesh(
        core_axis_name="c", subcore_axis_name="s", num_cores=NC, num_subcores=NS
    )
    grid_kernel = pl.kernel(
        functools.partial(_gather_body, steps),
        out_type=jax.ShapeDtypeStruct((n, D_MODEL), jnp.float32),
        mesh=mesh,
        scratch_types=[
            pltpu.VMEM((steps, CHUNK), jnp.int32),
            pltpu.VMEM((NBUF, CHUNK, D_MODEL), jnp.float32),
        ]
        + [pltpu.SemaphoreType.DMA] * (2 * NBUF),
    )
    return grid_kernel(table, idx.reshape(NW, steps, CHUNK))


def kernel(x, pe):
    b, l = x.shape
    n = b * l
    table = pe.reshape(MAX_LEN, D_MODEL)
    idx = x.reshape(-1).astype(jnp.int32)
    out = _sc_gather(table, idx, n)
    return out.reshape(b, l, D_MODEL)
